# baseline (speedup 1.0000x reference)
import functools

import jax
import jax.numpy as jnp
from jax import lax
from jax.experimental import pallas as pl
from jax.experimental.pallas import tpu as pltpu

_BLK = 2048


@functools.lru_cache(maxsize=None)
def _make_tc_gather(V, D, B):
    n_blocks = B // _BLK

    def body(idx_ref, table_ref, out_ref, sem):
        i = pl.program_id(0)

        def fire(j, carry):
            for u in range(8):
                row = idx_ref[0, 0, j * 8 + u]
                pltpu.make_async_copy(
                    table_ref.at[row], out_ref.at[i * _BLK + j * 8 + u], sem
                ).start()
            return carry

        lax.fori_loop(0, _BLK // 8, fire, 0)
        pltpu.make_async_copy(
            table_ref.at[pl.ds(0, _BLK)],
            out_ref.at[pl.ds(i * _BLK, _BLK)],
            sem,
        ).wait()

    return pl.pallas_call(
        body,
        grid=(n_blocks,),
        in_specs=[
            pl.BlockSpec((1, 1, _BLK), lambda i: (i, 0, 0),
                         memory_space=pltpu.SMEM),
            pl.BlockSpec(memory_space=pltpu.HBM),
        ],
        out_specs=pl.BlockSpec(memory_space=pltpu.HBM),
        out_shape=jax.ShapeDtypeStruct((B, D), jnp.float32),
        scratch_shapes=[pltpu.SemaphoreType.DMA],
    )


def kernel(nodes, table):
    (B,) = nodes.shape
    V, D = table.shape
    nodes3 = nodes.astype(jnp.int32).reshape(B // _BLK, 1, _BLK)
    return _make_tc_gather(V, D, B)(nodes3, table)


# speedup vs baseline: 1.2978x; 1.2978x over previous
import functools

import jax
import jax.numpy as jnp
from jax import lax
from jax.experimental import pallas as pl
from jax.experimental.pallas import tpu as pltpu
from jax.experimental.pallas import tpu_sc as plsc

_SC_SHARE = 10240
_BLK = 2048


@functools.lru_cache(maxsize=None)
def _make_sc_gather(V, D, B):
    info = plsc.get_sparse_core_info()
    NC, NS, L = info.num_cores, info.num_subcores, info.num_lanes
    NW = NC * NS
    assert B % (NW * L) == 0 and D % L == 0
    b_per_w = B // NW
    mesh = plsc.VectorSubcoreMesh(core_axis_name="c", subcore_axis_name="s")

    @functools.partial(
        pl.kernel,
        mesh=mesh,
        out_type=jax.ShapeDtypeStruct((B, D), jnp.float32),
        scratch_types=[
            pltpu.VMEM((b_per_w,), jnp.int32),
            pltpu.VMEM((b_per_w, D), jnp.float32),
            pltpu.SemaphoreType.DMA,
        ],
    )
    def gather_kernel(nodes_hbm, table_hbm, out_hbm, idx_v, rows_v, sem):
        wid = lax.axis_index("s") * NC + lax.axis_index("c")
        base = wid * b_per_w
        pltpu.sync_copy(nodes_hbm.at[pl.ds(base, b_per_w)], idx_v)

        def fire(j, carry):
            vec = idx_v[pl.ds(j * L, L)]
            for k in range(L):
                pltpu.async_copy(table_hbm.at[vec[k]], rows_v.at[j * L + k], sem)
            return carry

        lax.fori_loop(0, b_per_w // L, fire, 0)
        pltpu.make_async_copy(
            table_hbm.at[pl.ds(0, b_per_w)], rows_v, sem
        ).wait()
        pltpu.sync_copy(rows_v, out_hbm.at[pl.ds(base, b_per_w)])

    return gather_kernel


@functools.lru_cache(maxsize=None)
def _make_tc_gather(V, D, B):
    n_blocks = B // _BLK

    def body(idx_ref, table_ref, out_ref, sem):
        i = pl.program_id(0)

        def fire(j, carry):
            row = idx_ref[0, 0, j]
            pltpu.make_async_copy(
                table_ref.at[row], out_ref.at[i * _BLK + j], sem
            ).start()
            return carry

        lax.fori_loop(0, _BLK, fire, 0)
        pltpu.make_async_copy(
            table_ref.at[pl.ds(0, _BLK)],
            out_ref.at[pl.ds(i * _BLK, _BLK)],
            sem,
        ).wait()

    return pl.pallas_call(
        body,
        grid=(n_blocks,),
        in_specs=[
            pl.BlockSpec((1, 1, _BLK), lambda i: (i, 0, 0),
                         memory_space=pltpu.SMEM),
            pl.BlockSpec(memory_space=pltpu.HBM),
        ],
        out_specs=pl.BlockSpec(memory_space=pltpu.HBM),
        out_shape=jax.ShapeDtypeStruct((B, D), jnp.float32),
        scratch_shapes=[pltpu.SemaphoreType.DMA],
    )


def kernel(nodes, table):
    (B,) = nodes.shape
    V, D = table.shape
    nodes = nodes.astype(jnp.int32)
    n_sc = _SC_SHARE
    n_tc = B - n_sc
    out_sc = _make_sc_gather(V, D, n_sc)(nodes[:n_sc], table)
    out_tc = _make_tc_gather(V, D, n_tc)(
        nodes[n_sc:].reshape(n_tc // _BLK, 1, _BLK), table
    )
    return jnp.concatenate([out_sc, out_tc], axis=0)


# speedup vs baseline: 1.2990x; 1.0009x over previous
import functools

import jax
import jax.numpy as jnp
from jax import lax
from jax.experimental import pallas as pl
from jax.experimental.pallas import tpu as pltpu
from jax.experimental.pallas import tpu_sc as plsc

_SC_SHARE = 10240
_BLK = 2048


@functools.lru_cache(maxsize=None)
def _make_sc_gather(V, D, B):
    info = plsc.get_sparse_core_info()
    NC, NS, L = info.num_cores, info.num_subcores, info.num_lanes
    NW = NC * NS
    assert B % (NW * L) == 0 and D % L == 0
    b_per_w = B // NW
    mesh = plsc.VectorSubcoreMesh(core_axis_name="c", subcore_axis_name="s")

    @functools.partial(
        pl.kernel,
        mesh=mesh,
        out_type=jax.ShapeDtypeStruct((B, D), jnp.float32),
        scratch_types=[
            pltpu.VMEM((b_per_w,), jnp.int32),
            pltpu.VMEM((b_per_w, D), jnp.float32),
            pltpu.SemaphoreType.DMA,
        ],
        compiler_params=pltpu.CompilerParams(skip_device_barrier=True),
    )
    def gather_kernel(nodes_hbm, table_hbm, out_hbm, idx_v, rows_v, sem):
        wid = lax.axis_index("s") * NC + lax.axis_index("c")
        base = wid * b_per_w
        pltpu.sync_copy(nodes_hbm.at[pl.ds(base, b_per_w)], idx_v)

        def fire(j, carry):
            vec = idx_v[pl.ds(j * L, L)]
            for k in range(L):
                pltpu.async_copy(table_hbm.at[vec[k]], rows_v.at[j * L + k], sem)
            return carry

        lax.fori_loop(0, b_per_w // L, fire, 0)
        pltpu.make_async_copy(
            table_hbm.at[pl.ds(0, b_per_w)], rows_v, sem
        ).wait()
        pltpu.sync_copy(rows_v, out_hbm.at[pl.ds(base, b_per_w)])

    return gather_kernel


@functools.lru_cache(maxsize=None)
def _make_tc_gather(V, D, B):
    n_blocks = B // _BLK

    def body(idx_ref, table_ref, out_ref, sem):
        i = pl.program_id(0)

        def fire(j, carry):
            row = idx_ref[0, 0, j]
            pltpu.make_async_copy(
                table_ref.at[row], out_ref.at[i * _BLK + j], sem
            ).start()
            return carry

        lax.fori_loop(0, _BLK, fire, 0)
        pltpu.make_async_copy(
            table_ref.at[pl.ds(0, _BLK)],
            out_ref.at[pl.ds(i * _BLK, _BLK)],
            sem,
        ).wait()

    return pl.pallas_call(
        body,
        grid=(n_blocks,),
        in_specs=[
            pl.BlockSpec((1, 1, _BLK), lambda i: (i, 0, 0),
                         memory_space=pltpu.SMEM),
            pl.BlockSpec(memory_space=pltpu.HBM),
        ],
        out_specs=pl.BlockSpec(memory_space=pltpu.HBM),
        out_shape=jax.ShapeDtypeStruct((B, D), jnp.float32),
        scratch_shapes=[pltpu.SemaphoreType.DMA],
        compiler_params=pltpu.CompilerParams(skip_device_barrier=True),
    )


def kernel(nodes, table):
    (B,) = nodes.shape
    V, D = table.shape
    nodes = nodes.astype(jnp.int32)
    n_sc = _SC_SHARE
    n_tc = B - n_sc
    out_sc = _make_sc_gather(V, D, n_sc)(nodes[:n_sc], table)
    out_tc = _make_tc_gather(V, D, n_tc)(
        nodes[n_sc:].reshape(n_tc // _BLK, 1, _BLK), table
    )
    return jnp.concatenate([out_sc, out_tc], axis=0)


# speedup vs baseline: 1.6474x; 1.2683x over previous
import functools

import jax
import jax.numpy as jnp
from jax import lax
from jax.experimental import pallas as pl
from jax.experimental.pallas import tpu as pltpu
from jax.experimental.pallas import tpu_sc as plsc


@functools.lru_cache(maxsize=None)
def _make_gather(V, D, B):
    info = plsc.get_sparse_core_info()
    NC, NS = info.num_cores, info.num_subcores
    NW = NC * NS
    assert B % (8 * NW) == 0 and D % info.num_lanes == 0
    b_per_w = B // NW
    mesh = plsc.VectorSubcoreMesh(core_axis_name="c", subcore_axis_name="s")

    @functools.partial(
        pl.kernel,
        mesh=mesh,
        out_type=jax.ShapeDtypeStruct((B, D), jnp.float32),
        scratch_types=[
            pltpu.VMEM((b_per_w,), jnp.int32),
            pltpu.VMEM((b_per_w, D), jnp.float32),
            [pltpu.SemaphoreType.DMA] * 8,
        ],
    )
    def gather_kernel(nodes_hbm, table_hbm, out_hbm, idx_v, rows_v, sems):
        wid = lax.axis_index("s") * NC + lax.axis_index("c")
        base = wid * b_per_w
        pltpu.sync_copy(nodes_hbm.at[pl.ds(base, b_per_w)], idx_v)

        L = info.num_lanes

        def fire(j, carry):
            vec = idx_v[pl.ds(j * L, L)]
            for k in range(L):
                pltpu.async_copy(
                    table_hbm.at[vec[k]], rows_v.at[j * L + k], sems[k % 8]
                )
            return carry

        lax.fori_loop(0, b_per_w // L, fire, 0)
        n_per_sem = b_per_w // 8
        for k in range(8):
            pltpu.make_async_copy(
                table_hbm.at[pl.ds(0, n_per_sem)],
                rows_v.at[pl.ds(0, n_per_sem)],
                sems[k],
            ).wait()
        pltpu.sync_copy(rows_v, out_hbm.at[pl.ds(base, b_per_w)])

    return gather_kernel


def kernel(nodes, table):
    (B,) = nodes.shape
    V, D = table.shape
    return _make_gather(V, D, B)(nodes.astype(jnp.int32), table)
